# Initial kernel scaffold; baseline (speedup 1.0000x reference)
import jax
import jax.numpy as jnp
from jax.experimental import pallas as pl


def kernel(x, W):
    raise NotImplementedError("write your pallas kernel here")


# speedup vs baseline: 1.3727x; 1.3727x over previous
import functools

import jax
import jax.numpy as jnp
from jax import lax
from jax.experimental import pallas as pl
from jax.experimental.pallas import tpu as pltpu
from jax.experimental.pallas import tpu_sc as plsc

_NUM_IMG = 100000
_OUT_CH = 64
_BATCH = 1024

_NC = 2
_NS = 16
_NW = _NC * _NS
_BLOC = _BATCH // _NW
_LANES = 16

_mesh = plsc.VectorSubcoreMesh(core_axis_name="c", subcore_axis_name="s")


@functools.partial(
    pl.kernel,
    mesh=_mesh,
    out_type=jax.ShapeDtypeStruct((_BATCH, _OUT_CH), jnp.float32),
    scratch_types=[
        pltpu.VMEM((_BLOC,), jnp.int32),
        pltpu.VMEM((_BLOC, _OUT_CH), jnp.int32),
        pltpu.VMEM((_BLOC, _OUT_CH), jnp.float32),
        pltpu.SemaphoreType.DMA,
    ],
)
def _gather_kernel(x_hbm, w_hbm, out_hbm, x_v, idx_v, rows_v, sem):
    wid = lax.axis_index("s") * _NC + lax.axis_index("c")
    base = wid * _BLOC
    pltpu.sync_copy(x_hbm.at[pl.ds(base, _BLOC)], x_v)

    lane = lax.iota(jnp.int32, _LANES)
    cvec = lane * _NUM_IMG
    for g in range(_BLOC // _LANES):
        xv = x_v[pl.ds(g * _LANES, _LANES)]
        for j in range(_LANES):
            b = g * _LANES + j
            xs = xv[j]
            for cb in range(_OUT_CH // _LANES):
                idx_v[b, pl.ds(cb * _LANES, _LANES)] = cvec + (
                    xs + cb * _LANES * _NUM_IMG
                )

    copies = [
        pltpu.async_copy(w_hbm.at[idx_v.at[b]], rows_v.at[b], sem)
        for b in range(_BLOC)
    ]
    for cp in copies:
        cp.wait()

    pltpu.sync_copy(rows_v, out_hbm.at[pl.ds(base, _BLOC)])


def kernel(x, W):
    xi = x.astype(jnp.int32)
    w_flat = W.reshape(-1)
    y = _gather_kernel(xi, w_flat)
    return y[:, :, None, None]


# speedup vs baseline: 1.7825x; 1.2985x over previous
import functools

import jax
import jax.numpy as jnp
from jax import lax
from jax.experimental import pallas as pl
from jax.experimental.pallas import tpu as pltpu
from jax.experimental.pallas import tpu_sc as plsc

_NUM_IMG = 100000
_OUT_CH = 64
_BATCH = 1024

_NC = 2
_NS = 16
_NW = _NC * _NS
_BLOC = _BATCH // _NW
_LANES = 16
_TILE = 128
_NBUF = 8

_mesh = plsc.VectorSubcoreMesh(core_axis_name="c", subcore_axis_name="s")


@functools.partial(
    pl.kernel,
    mesh=_mesh,
    out_type=jax.ShapeDtypeStruct((_BATCH, _OUT_CH), jnp.float32),
    compiler_params=pltpu.CompilerParams(needs_layout_passes=False),
    scratch_types=[
        pltpu.VMEM((_BLOC,), jnp.int32),
        pltpu.VMEM((_OUT_CH,), jnp.int32),
        pltpu.VMEM((_NBUF, _OUT_CH, _TILE), jnp.float32),
        pltpu.VMEM((_BLOC, _OUT_CH), jnp.float32),
        pltpu.SemaphoreType.DMA,
    ],
)
def _gather_kernel(x_hbm, w_hbm, out_hbm, x_v, ch_v, blk_v, rows_v, sem):
    wid = lax.axis_index("s") * _NC + lax.axis_index("c")
    base = wid * _BLOC
    pltpu.sync_copy(x_hbm.at[pl.ds(base, _BLOC)], x_v)

    lane = lax.iota(jnp.int32, _LANES)
    for cb in range(_OUT_CH // _LANES):
        ch_v[pl.ds(cb * _LANES, _LANES)] = lane + cb * _LANES

    cols = []
    starts = []
    for g in range(_BLOC // _LANES):
        xv = x_v[pl.ds(g * _LANES, _LANES)]
        for j in range(_LANES):
            xs = xv[j]
            col = xs & (_TILE - 1)
            cols.append(col)
            starts.append(pl.multiple_of(xs - col, _TILE))

    def _fire(b):
        return pltpu.async_copy(
            w_hbm.at[ch_v, pl.ds(starts[b], _TILE)],
            blk_v.at[b % _NBUF],
            sem,
        )

    copies = [_fire(b) for b in range(_NBUF)]
    for b in range(_BLOC):
        copies[b].wait()
        colvec = lane * 0 + cols[b]
        for cb in range(_OUT_CH // _LANES):
            vals = plsc.load_gather(
                blk_v.at[b % _NBUF], [lane + cb * _LANES, colvec]
            )
            rows_v[b, pl.ds(cb * _LANES, _LANES)] = vals
        if b + _NBUF < _BLOC:
            copies.append(_fire(b + _NBUF))

    pltpu.sync_copy(rows_v, out_hbm.at[pl.ds(base, _BLOC)])


def kernel(x, W):
    xi = x.astype(jnp.int32)
    y = _gather_kernel(xi, W)
    return y[:, :, None, None]


# speedup vs baseline: 1.9922x; 1.1176x over previous
import functools

import jax
import jax.numpy as jnp
from jax import lax
from jax.experimental import pallas as pl
from jax.experimental.pallas import tpu as pltpu
from jax.experimental.pallas import tpu_sc as plsc

_NUM_IMG = 100000
_OUT_CH = 64
_BATCH = 1024

_NC = 2
_NS = 16
_NW = _NC * _NS
_BLOC = _BATCH // _NW
_LANES = 16
_TILE = 128
_RND = 4
_NROUNDS = _BLOC // _RND
_NBUF = 2 * _RND

_mesh = plsc.VectorSubcoreMesh(core_axis_name="c", subcore_axis_name="s")


@functools.partial(
    pl.kernel,
    mesh=_mesh,
    out_type=jax.ShapeDtypeStruct((_BATCH, _OUT_CH), jnp.float32),
    compiler_params=pltpu.CompilerParams(needs_layout_passes=False),
    scratch_types=[
        pltpu.VMEM((_BLOC + _LANES,), jnp.int32),
        pltpu.VMEM((_OUT_CH,), jnp.int32),
        pltpu.VMEM((_NBUF, _OUT_CH, _TILE), jnp.float32),
        pltpu.VMEM((_BLOC, _OUT_CH), jnp.float32),
        pltpu.SemaphoreType.DMA,
        pltpu.SemaphoreType.DMA,
    ],
)
def _gather_kernel(x_hbm, w_hbm, out_hbm, x_v, ch_v, blk_v, rows_v, sem_a, sem_b):
    wid = lax.axis_index("s") * _NC + lax.axis_index("c")
    base = wid * _BLOC
    pltpu.sync_copy(x_hbm.at[pl.ds(base, _BLOC)], x_v.at[pl.ds(0, _BLOC)])

    lane = lax.iota(jnp.int32, _LANES)
    for cb in range(_OUT_CH // _LANES):
        ch_v[pl.ds(cb * _LANES, _LANES)] = lane + cb * _LANES

    def _fire(xs, slot, sem):
        col = xs & (_TILE - 1)
        start = pl.multiple_of(xs - col, _TILE)
        pltpu.async_copy(
            w_hbm.at[ch_v, pl.ds(start, _TILE)], blk_v.at[slot], sem
        )

    def _wait(slot, sem):
        pltpu.make_async_copy(
            w_hbm.at[ch_v, pl.ds(0, _TILE)], blk_v.at[slot], sem
        ).wait()

    def _extract(b, slot, xs):
        colvec = lane * 0 + (xs & (_TILE - 1))
        for cb in range(_OUT_CH // _LANES):
            vals = plsc.load_gather(
                blk_v.at[slot], [lane + cb * _LANES, colvec]
            )
            rows_v[b, pl.ds(cb * _LANES, _LANES)] = vals

    xv0 = x_v[pl.ds(0, _LANES)]
    for j in range(_RND):
        _fire(xv0[j], j, sem_a)
    for j in range(_RND):
        _fire(xv0[_RND + j], _RND + j, sem_b)

    def _round(r, bank, sem):
        b0 = r * _RND
        xv = x_v[pl.ds(b0, _LANES)]

        for j in range(_RND):
            _wait(bank * _RND + j, sem)
        for j in range(_RND):
            _extract(b0 + j, bank * _RND + j, xv[j])

        @pl.when(r < _NROUNDS - 2)
        def _refill():
            for j in range(_RND):
                _fire(xv[2 * _RND + j], bank * _RND + j, sem)

    def _body(k, carry):
        r = 2 * k
        _round(r, 0, sem_a)
        _round(r + 1, 1, sem_b)
        return carry

    lax.fori_loop(0, _NROUNDS // 2, _body, 0)

    pltpu.sync_copy(rows_v, out_hbm.at[pl.ds(base, _BLOC)])


def kernel(x, W):
    xi = x.astype(jnp.int32)
    y = _gather_kernel(xi, W)
    return y[:, :, None, None]


# speedup vs baseline: 2.0820x; 1.0451x over previous
import functools

import jax
import jax.numpy as jnp
from jax import lax
from jax.experimental import pallas as pl
from jax.experimental.pallas import tpu as pltpu
from jax.experimental.pallas import tpu_sc as plsc

_NUM_IMG = 100000
_OUT_CH = 64
_BATCH = 1024

_NC = 2
_NS = 16
_NW = _NC * _NS
_BLOC = _BATCH // _NW
_LANES = 16
_TILE = 128
_RND = 4
_NROUNDS = _BLOC // _RND
_NBANK = 3
_NBUF = _NBANK * _RND

_mesh = plsc.VectorSubcoreMesh(core_axis_name="c", subcore_axis_name="s")


@functools.partial(
    pl.kernel,
    mesh=_mesh,
    out_type=jax.ShapeDtypeStruct((_BATCH, _OUT_CH), jnp.float32),
    compiler_params=pltpu.CompilerParams(needs_layout_passes=False),
    scratch_types=[
        pltpu.VMEM((_BLOC + _LANES,), jnp.int32),
        pltpu.VMEM((_OUT_CH,), jnp.int32),
        pltpu.VMEM((_NBUF, _OUT_CH, _TILE), jnp.float32),
        pltpu.VMEM((_BLOC, _OUT_CH), jnp.float32),
        pltpu.SemaphoreType.DMA,
    ],
)
def _gather_kernel(x_hbm, w_hbm, out_hbm, x_v, ch_v, blk_v, rows_v, sem):
    wid = lax.axis_index("s") * _NC + lax.axis_index("c")
    base = wid * _BLOC
    pltpu.sync_copy(x_hbm.at[pl.ds(base, _BLOC)], x_v.at[pl.ds(0, _BLOC)])

    lane = lax.iota(jnp.int32, _LANES)
    for cb in range(_OUT_CH // _LANES):
        ch_v[pl.ds(cb * _LANES, _LANES)] = lane + cb * _LANES

    def _fire(xs, slot):
        col = xs & (_TILE - 1)
        start = pl.multiple_of(xs - col, _TILE)
        pltpu.async_copy(
            w_hbm.at[ch_v, pl.ds(start, _TILE)], blk_v.at[slot], sem
        )

    def _wait(slot):
        pltpu.make_async_copy(
            w_hbm.at[ch_v, pl.ds(0, _TILE)], blk_v.at[slot], sem
        ).wait()

    def _extract(b, slot, xs):
        colvec = lane * 0 + (xs & (_TILE - 1))
        for cb in range(_OUT_CH // _LANES):
            vals = plsc.load_gather(
                blk_v.at[slot], [lane + cb * _LANES, colvec]
            )
            rows_v[b, pl.ds(cb * _LANES, _LANES)] = vals

    xv0 = x_v[pl.ds(0, _LANES)]
    for j in range(_RND):
        _fire(xv0[j], j)
    for j in range(_RND):
        _fire(xv0[_RND + j], _RND + j)

    def _body(r, carry):
        b0 = r * _RND
        slot0 = lax.rem(r, _NBANK) * _RND
        nslot0 = lax.rem(r + 2, _NBANK) * _RND
        xv = x_v[pl.ds(b0, _LANES)]

        for j in range(_RND):
            _wait(slot0 + j)

        @pl.when(r < _NROUNDS - 2)
        def _refill():
            for j in range(_RND):
                _fire(xv[2 * _RND + j], nslot0 + j)

        for j in range(_RND):
            _extract(b0 + j, slot0 + j, xv[j])
        return carry

    lax.fori_loop(0, _NROUNDS, _body, 0)

    pltpu.sync_copy(rows_v, out_hbm.at[pl.ds(base, _BLOC)])


def kernel(x, W):
    xi = x.astype(jnp.int32)
    y = _gather_kernel(xi, W)
    return y[:, :, None, None]


# speedup vs baseline: 2.0822x; 1.0001x over previous
import functools

import jax
import jax.numpy as jnp
from jax import lax
from jax.experimental import pallas as pl
from jax.experimental.pallas import tpu as pltpu
from jax.experimental.pallas import tpu_sc as plsc

_NUM_IMG = 100000
_OUT_CH = 64
_BATCH = 1024

_NC = 2
_NS = 16
_NW = _NC * _NS
_BLOC = _BATCH // _NW
_LANES = 16
_TILE = 128
_RND = 4
_NROUNDS = _BLOC // _RND
_NBANK = 3
_NBUF = _NBANK * _RND

_mesh = plsc.VectorSubcoreMesh(core_axis_name="c", subcore_axis_name="s")


@functools.partial(
    pl.kernel,
    mesh=_mesh,
    out_type=jax.ShapeDtypeStruct((_BATCH, _OUT_CH), jnp.float32),
    compiler_params=pltpu.CompilerParams(needs_layout_passes=False),
    scratch_types=[
        pltpu.VMEM((_BLOC + _LANES,), jnp.int32),
        pltpu.VMEM((_OUT_CH,), jnp.int32),
        pltpu.VMEM((_NBUF, _OUT_CH, _TILE), jnp.float32),
        pltpu.VMEM((_BLOC, _OUT_CH), jnp.float32),
        pltpu.SemaphoreType.DMA,
        pltpu.SemaphoreType.DMA,
    ],
)
def _gather_kernel(x_hbm, w_hbm, out_hbm, x_v, ch_v, blk_v, rows_v, sem, wsem):
    wid = lax.axis_index("s") * _NC + lax.axis_index("c")
    base = wid * _BLOC
    pltpu.sync_copy(x_hbm.at[pl.ds(base, _BLOC)], x_v.at[pl.ds(0, _BLOC)])

    lane = lax.iota(jnp.int32, _LANES)
    for cb in range(_OUT_CH // _LANES):
        ch_v[pl.ds(cb * _LANES, _LANES)] = lane + cb * _LANES

    def _fire(xs, slot):
        col = xs & (_TILE - 1)
        start = pl.multiple_of(xs - col, _TILE)
        pltpu.async_copy(
            w_hbm.at[ch_v, pl.ds(start, _TILE)], blk_v.at[slot], sem
        )

    def _wait(slot):
        pltpu.make_async_copy(
            w_hbm.at[pl.ds(0, _OUT_CH), pl.ds(0, _TILE)], blk_v.at[slot], sem
        ).wait()

    def _extract(b, slot, xs):
        colvec = lane * 0 + (xs & (_TILE - 1))
        for cb in range(_OUT_CH // _LANES):
            vals = plsc.load_gather(
                blk_v.at[slot], [lane + cb * _LANES, colvec]
            )
            rows_v[b, pl.ds(cb * _LANES, _LANES)] = vals

    xv0 = x_v[pl.ds(0, _LANES)]
    for j in range(_RND):
        _fire(xv0[j], j)
    for j in range(_RND):
        _fire(xv0[_RND + j], _RND + j)

    def _body(r, carry):
        b0 = r * _RND
        slot0 = lax.rem(r, _NBANK) * _RND
        nslot0 = lax.rem(r + 2, _NBANK) * _RND
        xv = x_v[pl.ds(b0, _LANES)]

        for j in range(_RND):
            _wait(slot0 + j)

        @pl.when(r < _NROUNDS - 2)
        def _refill():
            for j in range(_RND):
                _fire(xv[2 * _RND + j], nslot0 + j)

        for j in range(_RND):
            _extract(b0 + j, slot0 + j, xv[j])

        @pl.when(lax.rem(r, 2) == 1)
        def _flush():
            w0 = b0 - _RND
            pltpu.async_copy(
                rows_v.at[pl.ds(w0, 2 * _RND)],
                out_hbm.at[pl.ds(pl.multiple_of(base + w0, 8), 2 * _RND)],
                wsem,
            )
        return carry

    lax.fori_loop(0, _NROUNDS, _body, 0)

    for _ in range(_NROUNDS // 2):
        pltpu.make_async_copy(
            rows_v.at[pl.ds(0, 2 * _RND)],
            out_hbm.at[pl.ds(base, 2 * _RND)],
            wsem,
        ).wait()


def kernel(x, W):
    xi = x.astype(jnp.int32)
    y = _gather_kernel(xi, W)
    return y[:, :, None, None]
